# baseline (speedup 1.0000x reference)
import functools

import jax
import jax.numpy as jnp
from jax import lax
from jax.experimental import pallas as pl
from jax.experimental.pallas import tpu as pltpu
from jax.experimental.pallas import tpu_sc as plsc

N = 100000
DIM_IN = 128
DIM_PE = 32
DIM_H = 96

NUM_WORKERS = 32
B_PER_W = 3200
N_SUB = 8
BLOCK_ROWS = 10000


def _sc_gather(z1d, table):
    mesh = plsc.VectorSubcoreMesh(core_axis_name="c", subcore_axis_name="s")

    @functools.partial(
        pl.kernel,
        out_type=jax.ShapeDtypeStruct((N, DIM_PE), jnp.float32),
        mesh=mesh,
        scratch_types=[
            pltpu.VMEM((B_PER_W,), jnp.int32),
            pltpu.VMEM((B_PER_W, DIM_PE), jnp.float32),
        ]
        + [pltpu.SemaphoreType.DMA] * N_SUB
        + [pltpu.SemaphoreType.DMA],
        compiler_params=pltpu.CompilerParams(use_tc_tiling_on_sc=False),
    )
    def k(z_hbm, table_hbm, out_hbm, idx_v, rows_v, *sems):
        gsems, ssem = sems[:N_SUB], sems[N_SUB]
        wid = lax.axis_index("s") * 2 + lax.axis_index("c")
        base = jnp.minimum(wid * B_PER_W, N - B_PER_W)
        pltpu.sync_copy(z_hbm.at[pl.ds(base, B_PER_W)], idx_v)

        SUB = B_PER_W // N_SUB
        handles = [
            pltpu.async_copy(
                table_hbm.at[idx_v.at[pl.ds(i * SUB, SUB)]],
                rows_v.at[pl.ds(i * SUB, SUB)],
                gsems[i],
            )
            for i in range(N_SUB)
        ]
        for i in range(N_SUB):
            handles[i].wait()
            pltpu.async_copy(
                rows_v.at[pl.ds(i * SUB, SUB)],
                out_hbm.at[pl.ds(base + i * SUB, SUB)],
                ssem,
            )
        pltpu.make_async_copy(out_hbm.at[pl.ds(base, B_PER_W)], rows_v, ssem).wait()

    return k(z1d, table)


def _tc_body(x_ref, emb_ref, w_ref, b_ref, out_ref):
    h = jnp.dot(x_ref[...], w_ref[...], preferred_element_type=jnp.float32)
    out_ref[...] = jnp.concatenate([h + b_ref[...], emb_ref[...]], axis=1)


def _tc_fused(x, z_emb, W, b2d):
    return pl.pallas_call(
        _tc_body,
        grid=(N // BLOCK_ROWS,),
        in_specs=[
            pl.BlockSpec((BLOCK_ROWS, DIM_IN), lambda i: (i, 0)),
            pl.BlockSpec((BLOCK_ROWS, DIM_PE), lambda i: (i, 0)),
            pl.BlockSpec((DIM_IN, DIM_H), lambda i: (0, 0)),
            pl.BlockSpec((1, DIM_H), lambda i: (0, 0)),
        ],
        out_specs=pl.BlockSpec((BLOCK_ROWS, DIM_IN), lambda i: (i, 0)),
        out_shape=jax.ShapeDtypeStruct((N, DIM_IN), jnp.float32),
    )(x, z_emb, W, b2d)


def kernel(x, z, table, W, b):
    z_emb = _sc_gather(z.astype(jnp.int32), table)
    return _tc_fused(x, z_emb, W, b.reshape(1, DIM_H))


# speedup vs baseline: 1.0357x; 1.0357x over previous
import functools

import jax
import jax.numpy as jnp
from jax import lax
from jax.experimental import pallas as pl
from jax.experimental.pallas import tpu as pltpu
from jax.experimental.pallas import tpu_sc as plsc

N = 100000
DIM_IN = 128
DIM_PE = 32
DIM_H = 96

NUM_WORKERS = 32
B_PER_W = 3200
N_SUB = 2
BLOCK_ROWS = 10000


def _sc_gather(z1d, table):
    mesh = plsc.VectorSubcoreMesh(core_axis_name="c", subcore_axis_name="s")

    @functools.partial(
        pl.kernel,
        out_type=jax.ShapeDtypeStruct((N, DIM_PE), jnp.float32),
        mesh=mesh,
        scratch_types=[
            pltpu.VMEM((B_PER_W,), jnp.int32),
            pltpu.VMEM((B_PER_W, DIM_PE), jnp.float32),
        ]
        + [pltpu.SemaphoreType.DMA] * N_SUB
        + [pltpu.SemaphoreType.DMA],
        compiler_params=pltpu.CompilerParams(use_tc_tiling_on_sc=False),
    )
    def k(z_hbm, table_hbm, out_hbm, idx_v, rows_v, *sems):
        gsems, ssem = sems[:N_SUB], sems[N_SUB]
        wid = lax.axis_index("s") * 2 + lax.axis_index("c")
        base = jnp.minimum(wid * B_PER_W, N - B_PER_W)
        pltpu.sync_copy(z_hbm.at[pl.ds(base, B_PER_W)], idx_v)

        SUB = B_PER_W // N_SUB
        handles = [
            pltpu.async_copy(
                table_hbm.at[idx_v.at[pl.ds(i * SUB, SUB)]],
                rows_v.at[pl.ds(i * SUB, SUB)],
                gsems[i],
            )
            for i in range(N_SUB)
        ]
        for i in range(N_SUB):
            handles[i].wait()
            pltpu.async_copy(
                rows_v.at[pl.ds(i * SUB, SUB)],
                out_hbm.at[pl.ds(base + i * SUB, SUB)],
                ssem,
            )
        pltpu.make_async_copy(out_hbm.at[pl.ds(base, B_PER_W)], rows_v, ssem).wait()

    return k(z1d, table)


def _tc_body(x_ref, emb_ref, w_ref, b_ref, out_ref):
    h = jnp.dot(x_ref[...], w_ref[...], preferred_element_type=jnp.float32)
    out_ref[...] = jnp.concatenate([h + b_ref[...], emb_ref[...]], axis=1)


def _tc_fused(x, z_emb, W, b2d):
    return pl.pallas_call(
        _tc_body,
        grid=(N // BLOCK_ROWS,),
        in_specs=[
            pl.BlockSpec((BLOCK_ROWS, DIM_IN), lambda i: (i, 0)),
            pl.BlockSpec((BLOCK_ROWS, DIM_PE), lambda i: (i, 0)),
            pl.BlockSpec((DIM_IN, DIM_H), lambda i: (0, 0)),
            pl.BlockSpec((1, DIM_H), lambda i: (0, 0)),
        ],
        out_specs=pl.BlockSpec((BLOCK_ROWS, DIM_IN), lambda i: (i, 0)),
        out_shape=jax.ShapeDtypeStruct((N, DIM_IN), jnp.float32),
    )(x, z_emb, W, b2d)


def kernel(x, z, table, W, b):
    z_emb = _sc_gather(z.astype(jnp.int32), table)
    return _tc_fused(x, z_emb, W, b.reshape(1, DIM_H))
